# Initial kernel scaffold; baseline (speedup 1.0000x reference)
import jax
import jax.numpy as jnp
from jax.experimental import pallas as pl


def kernel(x, w0, bias0, g0, beta0, w1, bias1, g1, beta1, w2, bias2, g2, beta2, w3, bias3, g3, beta3, w4, bias4, g4, beta4, w5, bias5, g5, beta5, w6, bias6, g6, beta6, w7, bias7, g7, beta7):
    raise NotImplementedError("write your pallas kernel here")


# speedup vs baseline: 1.9637x; 1.9637x over previous
import jax
import jax.numpy as jnp
from jax.experimental import pallas as pl
from jax.experimental.pallas import tpu as pltpu

_EPS = 1e-5


def _im2col_rows(xv, ntaps, maxpad):
    c_in, t = xv.shape
    rows = []
    for j in range(ntaps):
        d = j - maxpad
        if d == 0:
            rows.append(xv)
        elif d > 0:
            rows.append(jnp.concatenate(
                [xv[:, d:], jnp.zeros((c_in, d), xv.dtype)], axis=1))
        else:
            rows.append(jnp.concatenate(
                [jnp.zeros((c_in, -d), xv.dtype), xv[:, :t + d]], axis=1))
    return jnp.concatenate(rows, axis=0)


def _conv_block(x_ref, w, ntaps, maxpad):
    rows = x_ref.shape[0]
    cols = [_im2col_rows(x_ref[r].astype(jnp.bfloat16), ntaps, maxpad)
            for r in range(rows)]
    x2c = jnp.concatenate(cols, axis=1)
    return jnp.dot(w, x2c, preferred_element_type=jnp.float32)


def _make_stats_kernel(ntaps, maxpad):
    def _body(x_ref, w_ref, stats_ref):
        @pl.when(pl.program_id(1) == 0)
        def _init():
            stats_ref[...] = jnp.zeros_like(stats_ref)
        y = _conv_block(x_ref, w_ref[...], ntaps, maxpad)
        s1 = jnp.sum(y, axis=1, keepdims=True)
        s2 = jnp.sum(y * y, axis=1, keepdims=True)
        stats_ref[...] += jnp.concatenate([s1, s2], axis=1)
    return _body


def _make_apply_kernel(ntaps, maxpad):
    def _body(x_ref, w_ref, shift_ref, o_ref):
        rows, _, t = x_ref.shape
        y = _conv_block(x_ref, w_ref[...], ntaps, maxpad) + shift_ref[...]
        y = jnp.maximum(y, 0.0)
        for r in range(rows):
            yr = y[:, r * t:(r + 1) * t]
            o_ref[r] = jnp.concatenate(
                [yr[:, :1], jnp.maximum(yr[:, 1:], yr[:, :t - 1])], axis=1)
    return _body


def _block_rows(b_per_part, target):
    rows = min(target, b_per_part)
    while b_per_part % rows:
        rows -= 1
    return rows


def _conv_bank_forward(x, weights, gammas, betas):
    k = len(weights)
    ck, c_in, _ = weights[0].shape
    b, c_in_x, t = x.shape
    assert c_in_x == c_in
    c_total = k * ck
    maxpad = k // 2
    ntaps = k
    kdim = ntaps * c_in

    w_rows = []
    for bi in range(k):
        ks = bi + 1
        p = ks // 2
        w = weights[bi].astype(jnp.float32)
        wb = jnp.zeros((ck, ntaps, c_in), jnp.float32)
        wb = wb.at[:, maxpad - p:maxpad - p + ks, :].set(
            jnp.transpose(w, (0, 2, 1)))
        w_rows.append(wb.reshape(ck, kdim))
    w_full = jnp.concatenate(w_rows, axis=0)

    n_parts = 2 if b % 2 == 0 else 1
    b_pp = b // n_parts

    rows_s = _block_rows(b_pp, 4)
    n_in_s = b_pp // rows_s
    stats_parts = pl.pallas_call(
        _make_stats_kernel(ntaps, maxpad),
        out_shape=jax.ShapeDtypeStruct((n_parts, c_total, 2), jnp.float32),
        grid=(n_parts, n_in_s),
        in_specs=[
            pl.BlockSpec((rows_s, c_in, t), lambda p, i: (p * n_in_s + i, 0, 0)),
            pl.BlockSpec((c_total, kdim), lambda p, i: (0, 0)),
        ],
        out_specs=pl.BlockSpec((None, c_total, 2), lambda p, i: (p, 0, 0)),
        compiler_params=pltpu.CompilerParams(
            dimension_semantics=("parallel", "arbitrary"),
            vmem_limit_bytes=48 * 1024 * 1024),
    )(x, w_full.astype(jnp.bfloat16))

    stats = jnp.sum(stats_parts, axis=0)
    n = float(b * t)
    mean = stats[:, 0] / n
    var = jnp.maximum(stats[:, 1] / n - mean * mean, 0.0)
    gamma = jnp.concatenate([g.astype(jnp.float32) for g in gammas])
    beta = jnp.concatenate([bt.astype(jnp.float32) for bt in betas])
    scale = gamma * jax.lax.rsqrt(var + _EPS)
    shift = beta - mean * scale
    w_apply = (w_full * scale[:, None]).astype(jnp.bfloat16)

    rows_a = _block_rows(b_pp, 4)
    n_in_a = b_pp // rows_a
    out = pl.pallas_call(
        _make_apply_kernel(ntaps, maxpad),
        out_shape=jax.ShapeDtypeStruct((b, c_total, t), jnp.float32),
        grid=(n_parts, n_in_a),
        in_specs=[
            pl.BlockSpec((rows_a, c_in, t), lambda p, i: (p * n_in_a + i, 0, 0)),
            pl.BlockSpec((c_total, kdim), lambda p, i: (0, 0)),
            pl.BlockSpec((c_total, 1), lambda p, i: (0, 0)),
        ],
        out_specs=pl.BlockSpec((rows_a, c_total, t), lambda p, i: (p * n_in_a + i, 0, 0)),
        compiler_params=pltpu.CompilerParams(
            dimension_semantics=("parallel", "arbitrary"),
            vmem_limit_bytes=48 * 1024 * 1024),
    )(x, w_apply, shift[:, None])
    return out


def kernel(x,
           w0, bias0, g0, beta0,
           w1, bias1, g1, beta1,
           w2, bias2, g2, beta2,
           w3, bias3, g3, beta3,
           w4, bias4, g4, beta4,
           w5, bias5, g5, beta5,
           w6, bias6, g6, beta6,
           w7, bias7, g7, beta7):
    weights = [w0, w1, w2, w3, w4, w5, w6, w7]
    gammas = [g0, g1, g2, g3, g4, g5, g6, g7]
    betas = [beta0, beta1, beta2, beta3, beta4, beta5, beta6, beta7]
    return _conv_bank_forward(x, weights, gammas, betas)


# speedup vs baseline: 1.9665x; 1.0014x over previous
import jax
import jax.numpy as jnp
from jax.experimental import pallas as pl
from jax.experimental.pallas import tpu as pltpu

_EPS = 1e-5


def _im2col_rows(xv, ntaps, maxpad):
    c_in, t = xv.shape
    rows = []
    for j in range(ntaps):
        d = j - maxpad
        if d == 0:
            rows.append(xv)
        elif d > 0:
            rows.append(jnp.concatenate(
                [xv[:, d:], jnp.zeros((c_in, d), xv.dtype)], axis=1))
        else:
            rows.append(jnp.concatenate(
                [jnp.zeros((c_in, -d), xv.dtype), xv[:, :t + d]], axis=1))
    return jnp.concatenate(rows, axis=0)


def _make_fused_kernel(ntaps, maxpad, n_total):
    def _body(x_ref, w_ref, gb_ref, o_ref, stats_ref, wbf_ref, shift_ref):
        ph = pl.program_id(0)
        i = pl.program_id(1)
        rows, _, t = x_ref.shape

        @pl.when((ph == 0) & (i == 0))
        def _init():
            stats_ref[...] = jnp.zeros_like(stats_ref)
            wbf_ref[...] = w_ref[...].astype(jnp.bfloat16)

        @pl.when(ph == 0)
        def _stats():
            w = wbf_ref[...]
            s1 = None
            s2 = None
            for r in range(rows):
                x2c = _im2col_rows(x_ref[r].astype(jnp.bfloat16), ntaps, maxpad)
                y = jnp.dot(w, x2c, preferred_element_type=jnp.float32)
                p1 = jnp.sum(y, axis=1, keepdims=True)
                p2 = jnp.sum(y * y, axis=1, keepdims=True)
                s1 = p1 if s1 is None else s1 + p1
                s2 = p2 if s2 is None else s2 + p2
            stats_ref[...] += jnp.concatenate([s1, s2], axis=1)

        @pl.when((ph == 1) & (i == 0))
        def _fold():
            stats = stats_ref[...]
            mean = stats[:, 0:1] * (1.0 / n_total)
            var = jnp.maximum(stats[:, 1:2] * (1.0 / n_total) - mean * mean, 0.0)
            scale = gb_ref[:, 0:1] * jax.lax.rsqrt(var + _EPS)
            shift_ref[...] = gb_ref[:, 1:2] - mean * scale
            wbf_ref[...] = (w_ref[...] * scale).astype(jnp.bfloat16)

        @pl.when(ph == 1)
        def _apply():
            w = wbf_ref[...]
            shift = shift_ref[...]
            for r in range(rows):
                x2c = _im2col_rows(x_ref[r].astype(jnp.bfloat16), ntaps, maxpad)
                y = jnp.dot(w, x2c, preferred_element_type=jnp.float32)
                o_ref[r, :, 1:] = jnp.maximum(
                    jnp.maximum(y[:, 1:], y[:, :t - 1]) + shift, 0.0)
                o_ref[r, :, 0:1] = jnp.maximum(y[:, 0:1] + shift, 0.0)
    return _body


def _block_rows(b_total, target):
    rows = min(target, b_total)
    while b_total % rows:
        rows -= 1
    return rows


def _conv_bank_forward(x, weights, gammas, betas):
    k = len(weights)
    ck, c_in, _ = weights[0].shape
    b, c_in_x, t = x.shape
    assert c_in_x == c_in
    c_total = k * ck
    maxpad = k // 2
    ntaps = k
    kdim = ntaps * c_in

    w_rows = []
    for bi in range(k):
        ks = bi + 1
        p = ks // 2
        w = weights[bi].astype(jnp.float32)
        wb = jnp.zeros((ck, ntaps, c_in), jnp.float32)
        wb = wb.at[:, maxpad - p:maxpad - p + ks, :].set(
            jnp.transpose(w, (0, 2, 1)))
        w_rows.append(wb.reshape(ck, kdim))
    w_full = jnp.concatenate(w_rows, axis=0)

    gamma = jnp.concatenate([g.astype(jnp.float32) for g in gammas])
    beta = jnp.concatenate([bt.astype(jnp.float32) for bt in betas])
    gb = jnp.stack([gamma, beta], axis=1)

    rows = _block_rows(b, 4)
    n_blk = b // rows
    out = pl.pallas_call(
        _make_fused_kernel(ntaps, maxpad, float(b * t)),
        out_shape=jax.ShapeDtypeStruct((b, c_total, t), jnp.float32),
        grid=(2, n_blk),
        in_specs=[
            pl.BlockSpec((rows, c_in, t), lambda ph, i: (i, 0, 0)),
            pl.BlockSpec((c_total, kdim), lambda ph, i: (0, 0)),
            pl.BlockSpec((c_total, 2), lambda ph, i: (0, 0)),
        ],
        out_specs=pl.BlockSpec((rows, c_total, t), lambda ph, i: (i * ph, 0, 0)),
        scratch_shapes=[
            pltpu.VMEM((c_total, 2), jnp.float32),
            pltpu.VMEM((c_total, kdim), jnp.bfloat16),
            pltpu.VMEM((c_total, 1), jnp.float32),
        ],
        compiler_params=pltpu.CompilerParams(
            dimension_semantics=("arbitrary", "arbitrary"),
            vmem_limit_bytes=48 * 1024 * 1024),
    )(x, w_full, gb)
    return out


def kernel(x,
           w0, bias0, g0, beta0,
           w1, bias1, g1, beta1,
           w2, bias2, g2, beta2,
           w3, bias3, g3, beta3,
           w4, bias4, g4, beta4,
           w5, bias5, g5, beta5,
           w6, bias6, g6, beta6,
           w7, bias7, g7, beta7):
    weights = [w0, w1, w2, w3, w4, w5, w6, w7]
    gammas = [g0, g1, g2, g3, g4, g5, g6, g7]
    betas = [beta0, beta1, beta2, beta3, beta4, beta5, beta6, beta7]
    return _conv_bank_forward(x, weights, gammas, betas)


# speedup vs baseline: 2.5715x; 1.3077x over previous
import jax
import jax.numpy as jnp
from jax.experimental import pallas as pl
from jax.experimental.pallas import tpu as pltpu

_EPS = 1e-5
_MCHUNK = 256


def _im2col_rows(xv, ntaps, maxpad):
    c_in, t = xv.shape
    rows = []
    for j in range(ntaps):
        d = j - maxpad
        if d == 0:
            rows.append(xv)
        elif d > 0:
            rows.append(jnp.concatenate(
                [xv[:, d:], jnp.zeros((c_in, d), xv.dtype)], axis=1))
        else:
            rows.append(jnp.concatenate(
                [jnp.zeros((c_in, -d), xv.dtype), xv[:, :t + d]], axis=1))
    return jnp.concatenate(rows, axis=0)


def _make_fused_kernel(ntaps, maxpad, n_total):
    pad = 2 * maxpad

    def _body(x_ref, w_ref, gb_ref, o_ref, c_ref, xe_ref, sx_ref,
              wbf_ref, shift_ref):
        ph = pl.program_id(0)
        i = pl.program_id(1)
        rows, c_in, t = x_ref.shape
        tp = t + 2 * pad

        @pl.when((ph == 0) & (i == 0))
        def _init():
            c_ref[...] = jnp.zeros_like(c_ref)
            sx_ref[...] = jnp.zeros_like(sx_ref)

        @pl.when(ph == 0)
        def _stats():
            z = jnp.zeros((c_in, pad), jnp.bfloat16)
            xp = jnp.concatenate(
                sum([[z, x_ref[r].astype(jnp.bfloat16), z]
                     for r in range(rows)], []), axis=1)
            lt = rows * tp
            deltas = list(range(-(ntaps - 1), ntaps))
            for g in range(0, len(deltas), 2):
                grp = deltas[g:g + 2]
                lo = max([0] + [-d for d in grp])
                hi = lt - max([0] + [d for d in grp])
                a = xp[:, lo:hi]
                b2 = jnp.concatenate(
                    [xp[:, lo + d:hi + d] for d in grp], axis=0)
                res = jax.lax.dot_general(
                    a, b2, (((1,), (1,)), ((), ())),
                    preferred_element_type=jnp.float32)
                for q, d in enumerate(grp):
                    r0 = (d + ntaps - 1) * c_in
                    c_ref[r0:r0 + c_in, :] += res[:, q * c_in:(q + 1) * c_in]
            sx_ref[...] += jnp.sum(xp, axis=1, keepdims=True,
                                   dtype=jnp.float32)
            zs = jnp.zeros((c_in, 2 * pad), jnp.bfloat16)
            xe_parts = []
            for r in range(rows):
                lo = r * tp
                sig = jnp.concatenate(
                    [xp[:, lo + t:lo + t + pad], zs,
                     xp[:, lo + pad:lo + 2 * pad]], axis=1)
                sig_t = jnp.transpose(sig)
                blks = []
                for j in range(ntaps):
                    d = j - maxpad
                    blks.append(jnp.concatenate(
                        [sig_t[pad + d:pad + maxpad + d, :],
                         sig_t[3 * pad - maxpad + d:3 * pad + d, :]], axis=0))
                xe_parts.append(jnp.concatenate(blks, axis=1))
            xe_step = jnp.concatenate(xe_parts, axis=0)
            epr = rows * 2 * maxpad
            xe_ref[pl.ds(i * epr, epr), :] = xe_step

        @pl.when((ph == 1) & (i == 0))
        def _fold():
            wf = w_ref[...]
            xe = xe_ref[...]
            ge = jax.lax.dot_general(xe, xe, (((0,), (0,)), ((), ())),
                                     preferred_element_type=jnp.float32)
            cfull = jnp.concatenate(
                [jnp.concatenate(
                    [c_ref[(dj2 - dj1 + ntaps - 1) * c_in:
                           (dj2 - dj1 + ntaps) * c_in, :]
                     for dj2 in range(ntaps)], axis=1)
                 for dj1 in range(ntaps)], axis=0)
            g = cfull - ge
            wg = jnp.dot(wf, g, preferred_element_type=jnp.float32)
            s2 = jnp.sum(wf * wg, axis=1, keepdims=True)
            ue = jnp.transpose(
                jnp.sum(xe, axis=0, keepdims=True, dtype=jnp.float32))
            u = jnp.concatenate([sx_ref[...]] * ntaps, axis=0) - ue
            s1 = jnp.dot(wf, u, preferred_element_type=jnp.float32)
            mean = s1 * (1.0 / n_total)
            var = jnp.maximum(s2 * (1.0 / n_total) - mean * mean, 0.0)
            scale = gb_ref[:, 0:1] * jax.lax.rsqrt(var + _EPS)
            shift_ref[...] = gb_ref[:, 1:2] - mean * scale
            wbf_ref[...] = (wf * scale).astype(jnp.bfloat16)

        @pl.when(ph == 1)
        def _apply():
            shift = shift_ref[...]
            ck = wbf_ref.shape[0] // ntaps
            c_in = x_ref.shape[1]
            x2c = jnp.concatenate(
                [_im2col_rows(x_ref[r].astype(jnp.bfloat16), ntaps, maxpad)
                 for r in range(rows)], axis=1)
            ys = []
            for bi in range(ntaps):
                ks = bi + 1
                j0 = maxpad - ks // 2
                ys.append(jnp.dot(
                    wbf_ref[bi * ck:(bi + 1) * ck,
                            j0 * c_in:(j0 + ks) * c_in],
                    x2c[j0 * c_in:(j0 + ks) * c_in, :],
                    preferred_element_type=jnp.float32))
            y = jnp.concatenate(ys, axis=0)
            for r in range(rows):
                yr = y[:, r * t:(r + 1) * t]
                o_ref[r, :, 1:] = jnp.maximum(
                    jnp.maximum(yr[:, 1:], yr[:, :t - 1]) + shift, 0.0)
                o_ref[r, :, 0:1] = jnp.maximum(yr[:, 0:1] + shift, 0.0)
    return _body


def _block_rows(b_total, target):
    rows = min(target, b_total)
    while b_total % rows:
        rows -= 1
    return rows


def _conv_bank_forward(x, weights, gammas, betas):
    k = len(weights)
    ck, c_in, _ = weights[0].shape
    b, c_in_x, t = x.shape
    assert c_in_x == c_in
    c_total = k * ck
    maxpad = k // 2
    ntaps = k
    kdim = ntaps * c_in

    w_rows = []
    for bi in range(k):
        ks = bi + 1
        p = ks // 2
        w = weights[bi].astype(jnp.float32)
        wb = jnp.zeros((ck, ntaps, c_in), jnp.float32)
        wb = wb.at[:, maxpad - p:maxpad - p + ks, :].set(
            jnp.transpose(w, (0, 2, 1)))
        w_rows.append(wb.reshape(ck, kdim))
    w_full = jnp.concatenate(w_rows, axis=0)

    gamma = jnp.concatenate([g.astype(jnp.float32) for g in gammas])
    beta = jnp.concatenate([bt.astype(jnp.float32) for bt in betas])
    gb = jnp.stack([gamma, beta], axis=1)

    rows = _block_rows(b, 4)
    n_blk = b // rows
    out = pl.pallas_call(
        _make_fused_kernel(ntaps, maxpad, float(b * t)),
        out_shape=jax.ShapeDtypeStruct((b, c_total, t), jnp.float32),
        grid=(2, n_blk),
        in_specs=[
            pl.BlockSpec((rows, c_in, t), lambda ph, i: (i, 0, 0)),
            pl.BlockSpec((c_total, kdim), lambda ph, i: (0, 0)),
            pl.BlockSpec((c_total, 2), lambda ph, i: (0, 0)),
        ],
        out_specs=pl.BlockSpec((rows, c_total, t), lambda ph, i: (i * ph, 0, 0)),
        scratch_shapes=[
            pltpu.VMEM(((2 * ntaps - 1) * c_in, c_in), jnp.float32),
            pltpu.VMEM((n_blk * rows * 2 * maxpad, kdim),
                       jnp.bfloat16),
            pltpu.VMEM((c_in, 1), jnp.float32),
            pltpu.VMEM((c_total, kdim), jnp.bfloat16),
            pltpu.VMEM((c_total, 1), jnp.float32),
        ],
        compiler_params=pltpu.CompilerParams(
            dimension_semantics=("arbitrary", "arbitrary"),
            vmem_limit_bytes=48 * 1024 * 1024),
    )(x, w_full, gb)
    return out


def kernel(x,
           w0, bias0, g0, beta0,
           w1, bias1, g1, beta1,
           w2, bias2, g2, beta2,
           w3, bias3, g3, beta3,
           w4, bias4, g4, beta4,
           w5, bias5, g5, beta5,
           w6, bias6, g6, beta6,
           w7, bias7, g7, beta7):
    weights = [w0, w1, w2, w3, w4, w5, w6, w7]
    gammas = [g0, g1, g2, g3, g4, g5, g6, g7]
    betas = [beta0, beta1, beta2, beta3, beta4, beta5, beta6, beta7]
    return _conv_bank_forward(x, weights, gammas, betas)


# speedup vs baseline: 2.6436x; 1.0280x over previous
import jax
import jax.numpy as jnp
from jax.experimental import pallas as pl
from jax.experimental.pallas import tpu as pltpu

_EPS = 1e-5
_MCHUNK = 256


def _im2col_rows(xv, ntaps, maxpad):
    c_in, t = xv.shape
    rows = []
    for j in range(ntaps):
        d = j - maxpad
        if d == 0:
            rows.append(xv)
        elif d > 0:
            rows.append(jnp.concatenate(
                [xv[:, d:], jnp.zeros((c_in, d), xv.dtype)], axis=1))
        else:
            rows.append(jnp.concatenate(
                [jnp.zeros((c_in, -d), xv.dtype), xv[:, :t + d]], axis=1))
    return jnp.concatenate(rows, axis=0)


def _make_fused_kernel(ntaps, maxpad, n_total):
    pad = 2 * maxpad

    def _body(x_ref, w_ref, gb_ref, o_ref, c_ref, xe_ref, sx_ref,
              wbf_ref, shift_ref):
        ph = pl.program_id(0)
        i = pl.program_id(1)
        rows, c_in, t = x_ref.shape
        tp = t + 2 * pad

        @pl.when((ph == 0) & (i == 0))
        def _init():
            c_ref[...] = jnp.zeros_like(c_ref)
            sx_ref[...] = jnp.zeros_like(sx_ref)

        @pl.when(ph == 0)
        def _stats():
            z = jnp.zeros((c_in, pad), jnp.bfloat16)
            xp = jnp.concatenate(
                sum([[z, x_ref[r].astype(jnp.bfloat16), z]
                     for r in range(rows)], []), axis=1)
            lt = rows * tp
            for d in range(-(ntaps - 1), ntaps):
                a = xp[:, max(0, -d):lt - max(0, d)]
                bmat = xp[:, max(0, d):lt + min(0, d)]
                r0 = (d + ntaps - 1) * c_in
                c_ref[r0:r0 + c_in, :] += jax.lax.dot_general(
                    a, bmat, (((1,), (1,)), ((), ())),
                    preferred_element_type=jnp.float32)
            sx_ref[...] += jnp.sum(xp, axis=1, keepdims=True,
                                   dtype=jnp.float32)
            zs = jnp.zeros((c_in, 2 * pad), jnp.bfloat16)
            xe_parts = []
            for r in range(rows):
                lo = r * tp
                sig = jnp.concatenate(
                    [xp[:, lo + t:lo + t + pad], zs,
                     xp[:, lo + pad:lo + 2 * pad]], axis=1)
                sig_t = jnp.transpose(sig)
                blks = []
                for j in range(ntaps):
                    d = j - maxpad
                    blks.append(jnp.concatenate(
                        [sig_t[pad + d:pad + maxpad + d, :],
                         sig_t[3 * pad - maxpad + d:3 * pad + d, :]], axis=0))
                xe_parts.append(jnp.concatenate(blks, axis=1))
            xe_step = jnp.concatenate(xe_parts, axis=0)
            epr = rows * 2 * maxpad
            xe_ref[pl.ds(i * epr, epr), :] = xe_step

        @pl.when((ph == 1) & (i == 0))
        def _fold():
            wf = w_ref[...]
            xe = xe_ref[...]
            ge = jax.lax.dot_general(xe, xe, (((0,), (0,)), ((), ())),
                                     preferred_element_type=jnp.float32)
            cfull = jnp.concatenate(
                [jnp.concatenate(
                    [c_ref[(dj2 - dj1 + ntaps - 1) * c_in:
                           (dj2 - dj1 + ntaps) * c_in, :]
                     for dj2 in range(ntaps)], axis=1)
                 for dj1 in range(ntaps)], axis=0)
            g = cfull - ge
            wg = jnp.dot(wf, g, preferred_element_type=jnp.float32)
            s2 = jnp.sum(wf * wg, axis=1, keepdims=True)
            ue = jnp.transpose(
                jnp.sum(xe, axis=0, keepdims=True, dtype=jnp.float32))
            u = jnp.concatenate([sx_ref[...]] * ntaps, axis=0) - ue
            s1 = jnp.dot(wf, u, preferred_element_type=jnp.float32)
            mean = s1 * (1.0 / n_total)
            var = jnp.maximum(s2 * (1.0 / n_total) - mean * mean, 0.0)
            scale = gb_ref[:, 0:1] * jax.lax.rsqrt(var + _EPS)
            shift_ref[...] = gb_ref[:, 1:2] - mean * scale
            wbf_ref[...] = (wf * scale).astype(jnp.bfloat16)

        @pl.when(ph == 1)
        def _apply():
            shift = shift_ref[...]
            ck = wbf_ref.shape[0] // ntaps
            c_in = x_ref.shape[1]
            x2c = jnp.concatenate(
                [_im2col_rows(x_ref[r].astype(jnp.bfloat16), ntaps, maxpad)
                 for r in range(rows)], axis=1)
            ys = []
            for bi in range(ntaps):
                ks = bi + 1
                j0 = maxpad - ks // 2
                ys.append(jnp.dot(
                    wbf_ref[bi * ck:(bi + 1) * ck,
                            j0 * c_in:(j0 + ks) * c_in],
                    x2c[j0 * c_in:(j0 + ks) * c_in, :],
                    preferred_element_type=jnp.float32))
            y = jnp.concatenate(ys, axis=0)
            for r in range(rows):
                yr = y[:, r * t:(r + 1) * t]
                o_ref[r, :, 1:] = jnp.maximum(
                    jnp.maximum(yr[:, 1:], yr[:, :t - 1]) + shift, 0.0)
                o_ref[r, :, 0:1] = jnp.maximum(yr[:, 0:1] + shift, 0.0)
    return _body


def _block_rows(b_total, target):
    rows = min(target, b_total)
    while b_total % rows:
        rows -= 1
    return rows


def _conv_bank_forward(x, weights, gammas, betas):
    k = len(weights)
    ck, c_in, _ = weights[0].shape
    b, c_in_x, t = x.shape
    assert c_in_x == c_in
    c_total = k * ck
    maxpad = k // 2
    ntaps = k
    kdim = ntaps * c_in

    w_rows = []
    for bi in range(k):
        ks = bi + 1
        p = ks // 2
        w = weights[bi].astype(jnp.float32)
        wb = jnp.zeros((ck, ntaps, c_in), jnp.float32)
        wb = wb.at[:, maxpad - p:maxpad - p + ks, :].set(
            jnp.transpose(w, (0, 2, 1)))
        w_rows.append(wb.reshape(ck, kdim))
    w_full = jnp.concatenate(w_rows, axis=0)

    gamma = jnp.concatenate([g.astype(jnp.float32) for g in gammas])
    beta = jnp.concatenate([bt.astype(jnp.float32) for bt in betas])
    gb = jnp.stack([gamma, beta], axis=1)

    rows = _block_rows(b, 4)
    n_blk = b // rows
    out = pl.pallas_call(
        _make_fused_kernel(ntaps, maxpad, float(b * t)),
        out_shape=jax.ShapeDtypeStruct((b, c_total, t), jnp.float32),
        grid=(2, n_blk),
        in_specs=[
            pl.BlockSpec((rows, c_in, t), lambda ph, i: (i, 0, 0)),
            pl.BlockSpec((c_total, kdim), lambda ph, i: (0, 0)),
            pl.BlockSpec((c_total, 2), lambda ph, i: (0, 0)),
        ],
        out_specs=pl.BlockSpec((rows, c_total, t), lambda ph, i: (i * ph, 0, 0)),
        scratch_shapes=[
            pltpu.VMEM(((2 * ntaps - 1) * c_in, c_in), jnp.float32),
            pltpu.VMEM((n_blk * rows * 2 * maxpad, kdim),
                       jnp.bfloat16),
            pltpu.VMEM((c_in, 1), jnp.float32),
            pltpu.VMEM((c_total, kdim), jnp.bfloat16),
            pltpu.VMEM((c_total, 1), jnp.float32),
        ],
        compiler_params=pltpu.CompilerParams(
            dimension_semantics=("arbitrary", "arbitrary"),
            vmem_limit_bytes=48 * 1024 * 1024),
    )(x, w_full, gb)
    return out


def kernel(x,
           w0, bias0, g0, beta0,
           w1, bias1, g1, beta1,
           w2, bias2, g2, beta2,
           w3, bias3, g3, beta3,
           w4, bias4, g4, beta4,
           w5, bias5, g5, beta5,
           w6, bias6, g6, beta6,
           w7, bias7, g7, beta7):
    weights = [w0, w1, w2, w3, w4, w5, w6, w7]
    gammas = [g0, g1, g2, g3, g4, g5, g6, g7]
    betas = [beta0, beta1, beta2, beta3, beta4, beta5, beta6, beta7]
    return _conv_bank_forward(x, weights, gammas, betas)
